# baseline (speedup 1.0000x reference)
import functools

import jax
import jax.numpy as jnp
from jax.experimental import pallas as pl

_NQ = 4
_K = 8192
_D = 32
_TB = 1024
_KB = 1024
_HALF = _K // 2


def _rvq_block_kernel(x_ref, cb_ref, qout_ref, idx_ref, loss_ref, *, n_tokens):
    step = pl.program_id(0)

    @pl.when(step == 0)
    def _init():
        loss_ref[...] = jnp.zeros_like(loss_ref)

    res = x_ref[...]
    qsum = jnp.zeros_like(res)
    idx_cols = []
    loss_parts = []
    inv_count = 1.0 / (n_tokens * _D)
    for q in range(_NQ):
        cb = cb_ref[q]
        c2 = jnp.sum(cb * cb, axis=1)
        r2 = jnp.sum(res * res, axis=1, keepdims=True)
        res_bf = res.astype(jnp.bfloat16)
        half_min = []
        half_idx = []
        for h in range(2):
            best_d = jnp.full((_TB, 1), jnp.inf, jnp.float32)
            best_i = jnp.zeros((_TB, 1), jnp.int32)
            for b in range(h * (_HALF // _KB), (h + 1) * (_HALF // _KB)):
                cblk = cb[b * _KB:(b + 1) * _KB, :]
                mm = jnp.dot(res_bf, cblk.astype(jnp.bfloat16).T,
                             preferred_element_type=jnp.float32)
                dist = r2 - 2.0 * mm + c2[b * _KB:(b + 1) * _KB][None, :]
                lmin = jnp.min(dist, axis=1, keepdims=True)
                iota = jax.lax.broadcasted_iota(jnp.int32, (_TB, _KB), 1)
                lidx = jnp.min(jnp.where(dist == lmin, iota, _K), axis=1,
                               keepdims=True) + b * _KB
                upd = lmin < best_d
                best_d = jnp.where(upd, lmin, best_d)
                best_i = jnp.where(upd, lidx, best_i)
            half_min.append(best_d)
            half_idx.append(best_i)
        q0 = half_min[0].astype(jnp.bfloat16).astype(jnp.float32)
        take1 = half_min[1] < q0
        best_i = jnp.where(take1, half_idx[1], half_idx[0])
        quant = jnp.zeros((_TB, _D), jnp.float32)
        for b in range(_K // _KB):
            iota = jax.lax.broadcasted_iota(jnp.int32, (_TB, _KB), 1)
            oh = jnp.where(iota == best_i - b * _KB, 1.0, 0.0)
            quant = quant + jnp.dot(oh, cb[b * _KB:(b + 1) * _KB, :],
                                    precision=jax.lax.Precision.HIGHEST)
        q_st = res + (quant - res)
        diff = res - q_st
        loss_parts.append((jnp.sum(diff * diff) * inv_count).reshape(1, 1))
        res = res - q_st
        qsum = qsum + q_st
        idx_cols.append(best_i)
    qout_ref[...] = qsum
    idx_ref[...] = jnp.concatenate(idx_cols, axis=1)
    loss_ref[...] += jnp.concatenate(loss_parts, axis=1)


def kernel(x, codebooks):
    B, N, D = x.shape
    T = B * N
    flat = x.reshape(T, D)

    grid = (T // _TB,)
    qout, idx, loss = pl.pallas_call(
        functools.partial(_rvq_block_kernel, n_tokens=T),
        grid=grid,
        in_specs=[
            pl.BlockSpec((_TB, _D), lambda i: (i, 0)),
            pl.BlockSpec((_NQ, _K, _D), lambda i: (0, 0, 0)),
        ],
        out_specs=[
            pl.BlockSpec((_TB, _D), lambda i: (i, 0)),
            pl.BlockSpec((_TB, _NQ), lambda i: (i, 0)),
            pl.BlockSpec((1, _NQ), lambda i: (0, 0)),
        ],
        out_shape=[
            jax.ShapeDtypeStruct((T, D), jnp.float32),
            jax.ShapeDtypeStruct((T, _NQ), jnp.int32),
            jax.ShapeDtypeStruct((1, _NQ), jnp.float32),
        ],
    )(flat, codebooks)
    return qout.reshape(x.shape), idx.reshape(B, N, _NQ), loss.reshape(_NQ)


# speedup vs baseline: 2.6673x; 2.6673x over previous
import functools

import jax
import jax.numpy as jnp
from jax.experimental import pallas as pl

_NQ = 4
_K = 8192
_D = 32
_TB = 1024
_KB = 1024
_HALF = _K // 2


def _rvq_block_kernel(x_ref, cb_ref, qout_ref, idx_ref, loss_ref, *, n_tokens):
    step = pl.program_id(0)

    @pl.when(step == 0)
    def _init():
        loss_ref[...] = jnp.zeros_like(loss_ref)

    res = x_ref[...]
    qsum = jnp.zeros_like(res)
    idx_cols = []
    loss_parts = []
    inv_count = 1.0 / (n_tokens * _D)
    iota = jax.lax.broadcasted_iota(jnp.int32, (_TB, _KB), 1)
    for q in range(_NQ):
        cb = cb_ref[q]
        c2 = jnp.sum(cb * cb, axis=1)
        r2 = jnp.sum(res * res, axis=1, keepdims=True)
        res_bf = res.astype(jnp.bfloat16)
        cb2_bf = (cb + cb).astype(jnp.bfloat16)
        cb_hi = cb.astype(jnp.bfloat16)
        rem1 = cb - cb_hi.astype(jnp.float32)
        cb_mid = rem1.astype(jnp.bfloat16)
        cb_lo = (rem1 - cb_mid.astype(jnp.float32)).astype(jnp.bfloat16)
        cb3 = jnp.concatenate(
            [cb_hi, cb_mid, cb_lo], axis=1)
        half_min = []
        half_idx = []
        for h in range(2):
            best_d = jnp.full((_TB, 1), jnp.inf, jnp.float32)
            best_i = jnp.zeros((_TB, 1), jnp.int32)
            for b in range(h * (_HALF // _KB), (h + 1) * (_HALF // _KB)):
                mm2 = jnp.dot(res_bf, cb2_bf[b * _KB:(b + 1) * _KB, :].T,
                              preferred_element_type=jnp.float32)
                dist = r2 - mm2 + c2[b * _KB:(b + 1) * _KB][None, :]
                lmin = jnp.min(dist, axis=1, keepdims=True)
                lidx = jnp.min(jnp.where(dist == lmin, iota, _K), axis=1,
                               keepdims=True) + b * _KB
                upd = lmin < best_d
                best_d = jnp.where(upd, lmin, best_d)
                best_i = jnp.where(upd, lidx, best_i)
            half_min.append(best_d)
            half_idx.append(best_i)
        q0 = half_min[0].astype(jnp.bfloat16).astype(jnp.float32)
        take1 = half_min[1] < q0
        best_i = jnp.where(take1, half_idx[1], half_idx[0])
        parts = jnp.zeros((_TB, 3 * _D), jnp.float32)
        for b in range(_K // _KB):
            oh = jnp.where(iota == best_i - b * _KB,
                           1.0, 0.0).astype(jnp.bfloat16)
            parts = parts + jnp.dot(oh, cb3[b * _KB:(b + 1) * _KB, :],
                                    preferred_element_type=jnp.float32)
        quant = (parts[:, :_D] + parts[:, _D:2 * _D]) + parts[:, 2 * _D:]
        q_st = res + (quant - res)
        diff = res - q_st
        loss_parts.append((jnp.sum(diff * diff) * inv_count).reshape(1, 1))
        res = res - q_st
        qsum = qsum + q_st
        idx_cols.append(best_i)
    qout_ref[...] = qsum
    idx_ref[...] = jnp.concatenate(idx_cols, axis=1)
    loss_ref[...] += jnp.concatenate(loss_parts, axis=1)


def kernel(x, codebooks):
    B, N, D = x.shape
    T = B * N
    flat = x.reshape(T, D)

    grid = (T // _TB,)
    qout, idx, loss = pl.pallas_call(
        functools.partial(_rvq_block_kernel, n_tokens=T),
        grid=grid,
        in_specs=[
            pl.BlockSpec((_TB, _D), lambda i: (i, 0)),
            pl.BlockSpec((_NQ, _K, _D), lambda i: (0, 0, 0)),
        ],
        out_specs=[
            pl.BlockSpec((_TB, _D), lambda i: (i, 0)),
            pl.BlockSpec((_TB, _NQ), lambda i: (i, 0)),
            pl.BlockSpec((1, _NQ), lambda i: (0, 0)),
        ],
        out_shape=[
            jax.ShapeDtypeStruct((T, D), jnp.float32),
            jax.ShapeDtypeStruct((T, _NQ), jnp.int32),
            jax.ShapeDtypeStruct((1, _NQ), jnp.float32),
        ],
    )(flat, codebooks)
    return qout.reshape(x.shape), idx.reshape(B, N, _NQ), loss.reshape(_NQ)
